# baseline (speedup 1.0000x reference)
import functools

import jax
import jax.numpy as jnp
from jax import lax
from jax.experimental import pallas as pl
from jax.experimental.pallas import tpu as pltpu
from jax.experimental.pallas import tpu_sc as plsc

V = 1000
BATCH = 1024
SEQ = 20
N_TOK = BATCH * SEQ
NC = 2
NS = 16
NW = NC * NS
SPW = BATCH // NW
TPW = SPW * SEQ
CHUNK = 2 * SEQ
NCHUNK = TPW // CHUNK


def _lse_body(tab_ref, lse_ref):
    x = tab_ref[...]
    m = jnp.max(x, axis=1)
    s = jnp.sum(jnp.exp(x - m[:, None]), axis=1)
    lse_ref[...] = jnp.log(s) + m


def _row_lse(table):
    return pl.pallas_call(
        _lse_body,
        out_shape=jax.ShapeDtypeStruct((V,), jnp.float32),
    )(table)


def _sc_body(table_hbm, idx_hbm, tgt_hbm, lse_hbm,
             out_hbm, part_hbm,
             idx_v, tgt_v, lse_v, rows_a, rows_b, part_v,
             gsem_a, gsem_b, ssem_a, ssem_b):
    wid = lax.axis_index("s") * NC + lax.axis_index("c")
    base = wid * TPW
    sbase = wid * SPW

    pltpu.sync_copy(idx_hbm.at[pl.ds(base, TPW)], idx_v.at[pl.ds(0, TPW)])
    pltpu.sync_copy(tgt_hbm.at[pl.ds(base, TPW)], tgt_v.at[pl.ds(0, TPW)])
    pltpu.sync_copy(lse_hbm, lse_v)

    lanes = lax.iota(jnp.int32, 16)
    idx_v[pl.ds(TPW, 16)] = jnp.zeros((16,), jnp.int32)
    tgt_v[pl.ds(TPW, 16)] = jnp.zeros((16,), jnp.int32)

    def step(c, acc):
        g = pltpu.async_copy(table_hbm.at[idx_v.at[pl.ds(c * CHUNK, CHUNK)]],
                             rows_a, gsem_a)
        g.wait()
        pltpu.sync_copy(rows_a.at[pl.ds(0, SEQ)], out_hbm.at[sbase + c * 2])
        pltpu.sync_copy(rows_a.at[pl.ds(SEQ, SEQ)],
                        out_hbm.at[sbase + c * 2 + 1])
        for j in range(3):
            off = c * CHUNK + j * 16
            idx16 = idx_v[pl.ds(off, 16)]
            tgt16 = tgt_v[pl.ds(off, 16)]
            rid = lanes + j * 16
            if j == 2:
                rid = jnp.minimum(rid, CHUNK - 1)
                idx16 = jnp.clip(idx16, 0, V - 1)
                tgt16 = jnp.clip(tgt16, 0, V - 1)
            tv = plsc.load_gather(rows_a, [rid, tgt16])
            lv = plsc.load_gather(lse_v, [idx16])
            contrib = lv - tv
            if j == 2:
                contrib = jnp.where(lanes < 8, contrib, 0.0)
            acc = acc + contrib
        return acc

    acc = lax.fori_loop(0, NCHUNK, step, jnp.zeros((16,), jnp.float32))
    part_v[...] = acc
    pltpu.sync_copy(part_v, part_hbm.at[pl.ds(wid * 16, 16)])


@functools.partial(
    pl.kernel,
    out_type=(jax.ShapeDtypeStruct((BATCH, SEQ, V), jnp.float32),
              jax.ShapeDtypeStruct((NW * 16,), jnp.float32)),
    mesh=plsc.VectorSubcoreMesh(core_axis_name="c", subcore_axis_name="s"),
    scratch_types=[
        pltpu.VMEM((TPW + 16,), jnp.int32),
        pltpu.VMEM((TPW + 16,), jnp.int32),
        pltpu.VMEM((V,), jnp.float32),
        pltpu.VMEM((CHUNK, V), jnp.float32),
        pltpu.VMEM((CHUNK, V), jnp.float32),
        pltpu.VMEM((16,), jnp.float32),
        pltpu.SemaphoreType.DMA,
        pltpu.SemaphoreType.DMA,
        pltpu.SemaphoreType.DMA,
        pltpu.SemaphoreType.DMA,
    ],
    compiler_params=pltpu.CompilerParams(use_tc_tiling_on_sc=False,
                                         needs_layout_passes=False),
)
def _sc_gather(*args):
    _sc_body(*args)


def kernel(token_embedding, idx, targets):
    lse = _row_lse(token_embedding)
    logits, partials = _sc_gather(
        token_embedding, idx.reshape(-1), targets.reshape(-1), lse)
    loss = jnp.sum(partials) / N_TOK
    return logits, loss


# speedup vs baseline: 1.0366x; 1.0366x over previous
import functools

import jax
import jax.numpy as jnp
from jax import lax
from jax.experimental import pallas as pl
from jax.experimental.pallas import tpu as pltpu
from jax.experimental.pallas import tpu_sc as plsc

V = 1000
BATCH = 1024
SEQ = 20
N_TOK = BATCH * SEQ
NC = 2
NS = 16
NW = NC * NS
SPW = BATCH // NW
TPW = SPW * SEQ
CHUNK = 2 * SEQ
NCHUNK = TPW // CHUNK


def _lse_body(tab_ref, lse_ref):
    x = tab_ref[...]
    m = jnp.max(x, axis=1)
    s = jnp.sum(jnp.exp(x - m[:, None]), axis=1)
    lse_ref[...] = jnp.log(s) + m


def _row_lse(table):
    return pl.pallas_call(
        _lse_body,
        out_shape=jax.ShapeDtypeStruct((V,), jnp.float32),
    )(table)


def _sc_body(table_hbm, idx_hbm, tgt_hbm, lse_hbm,
             out_hbm, part_hbm,
             idx_v, tgt_v, lse_v, rows_a, rows_b, part_v,
             gsem_a, gsem_b, ssem_a, ssem_b):
    wid = lax.axis_index("s") * NC + lax.axis_index("c")
    base = wid * TPW
    sbase = wid * SPW

    pltpu.sync_copy(idx_hbm.at[pl.ds(base, TPW)], idx_v.at[pl.ds(0, TPW)])
    pltpu.sync_copy(tgt_hbm.at[pl.ds(base, TPW)], tgt_v.at[pl.ds(0, TPW)])
    pltpu.sync_copy(lse_hbm, lse_v)

    lanes = lax.iota(jnp.int32, 16)
    idx_v[pl.ds(TPW, 16)] = jnp.zeros((16,), jnp.int32)
    tgt_v[pl.ds(TPW, 16)] = jnp.zeros((16,), jnp.int32)

    rows = (rows_a, rows_b)
    gsem = (gsem_a, gsem_b)
    ssem = (ssem_a, ssem_b)

    def gather(c):
        b = c % 2
        return pltpu.async_copy(
            table_hbm.at[idx_v.at[pl.ds(c * CHUNK, CHUNK)]], rows[b], gsem[b])

    def scatter(c):
        b = c % 2
        s0 = pltpu.async_copy(rows[b].at[pl.ds(0, SEQ)],
                              out_hbm.at[sbase + c * 2], ssem[b])
        s1 = pltpu.async_copy(rows[b].at[pl.ds(SEQ, SEQ)],
                              out_hbm.at[sbase + c * 2 + 1], ssem[b])
        return (s0, s1)

    def loss(c, acc):
        b = c % 2
        for j in range(3):
            off = c * CHUNK + j * 16
            idx16 = idx_v[pl.ds(off, 16)]
            tgt16 = tgt_v[pl.ds(off, 16)]
            rid = lanes + j * 16
            if j == 2:
                rid = jnp.minimum(rid, CHUNK - 1)
                idx16 = jnp.clip(idx16, 0, V - 1)
                tgt16 = jnp.clip(tgt16, 0, V - 1)
            tv = plsc.load_gather(rows[b], [rid, tgt16])
            lv = plsc.load_gather(lse_v, [idx16])
            contrib = lv - tv
            if j == 2:
                contrib = jnp.where(lanes < 8, contrib, 0.0)
            acc = acc + contrib
        return acc

    acc = jnp.zeros((16,), jnp.float32)
    gds = [None, None]
    sds = [None, None]
    for c in range(NCHUNK):
        b = c % 2
        if sds[b] is not None:
            sds[b][0].wait()
            sds[b][1].wait()
            sds[b] = None
        gds[b] = gather(c)
        if c >= 1:
            pb = (c - 1) % 2
            gds[pb].wait()
            sds[pb] = scatter(c - 1)
            acc = loss(c - 1, acc)
    lb = (NCHUNK - 1) % 2
    gds[lb].wait()
    sds[lb] = scatter(NCHUNK - 1)
    acc = loss(NCHUNK - 1, acc)
    for b in range(2):
        if sds[b] is not None:
            sds[b][0].wait()
            sds[b][1].wait()
    part_v[...] = acc
    pltpu.sync_copy(part_v, part_hbm.at[pl.ds(wid * 16, 16)])


@functools.partial(
    pl.kernel,
    out_type=(jax.ShapeDtypeStruct((BATCH, SEQ, V), jnp.float32),
              jax.ShapeDtypeStruct((NW * 16,), jnp.float32)),
    mesh=plsc.VectorSubcoreMesh(core_axis_name="c", subcore_axis_name="s"),
    scratch_types=[
        pltpu.VMEM((TPW + 16,), jnp.int32),
        pltpu.VMEM((TPW + 16,), jnp.int32),
        pltpu.VMEM((V,), jnp.float32),
        pltpu.VMEM((CHUNK, V), jnp.float32),
        pltpu.VMEM((CHUNK, V), jnp.float32),
        pltpu.VMEM((16,), jnp.float32),
        pltpu.SemaphoreType.DMA,
        pltpu.SemaphoreType.DMA,
        pltpu.SemaphoreType.DMA,
        pltpu.SemaphoreType.DMA,
    ],
    compiler_params=pltpu.CompilerParams(use_tc_tiling_on_sc=False,
                                         needs_layout_passes=False),
)
def _sc_gather(*args):
    _sc_body(*args)


def kernel(token_embedding, idx, targets):
    lse = _row_lse(token_embedding)
    logits, partials = _sc_gather(
        token_embedding, idx.reshape(-1), targets.reshape(-1), lse)
    loss = jnp.sum(partials) / N_TOK
    return logits, loss
